# baseline (device time: 19537 ns/iter reference)
import jax
import jax.numpy as jnp
from jax import lax
from jax.experimental import pallas as pl
from jax.experimental.pallas import tpu as pltpu


def kernel(x):
    m_per, n = x.shape
    n_half = n // 2

    def body(
        x_ref,
        out_ref,
        xv_peer,
        xv_mine,
        comm_ref,
        local_bf,
        in_sems,
        out_sem,
        send_sem,
        recv_sem,
    ):
        my_x = lax.axis_index("x")
        my_y = lax.axis_index("y")
        my_z = lax.axis_index("z")
        peer_z = 1 - my_z

        dma_peer = pltpu.make_async_copy(
            x_ref.at[:, pl.ds(peer_z * n_half, n_half)], xv_peer, in_sems.at[0]
        )
        dma_peer.start()
        dma_mine = pltpu.make_async_copy(
            x_ref.at[:, pl.ds(my_z * n_half, n_half)], xv_mine, in_sems.at[1]
        )
        dma_mine.start()

        barrier_sem = pltpu.get_barrier_semaphore()
        pl.semaphore_signal(
            barrier_sem,
            inc=1,
            device_id=(my_x, my_y, peer_z),
            device_id_type=pl.DeviceIdType.MESH,
        )
        pl.semaphore_wait(barrier_sem, 1)

        dma_peer.wait()
        comm_ref[:, :] = xv_peer[:, :].astype(jnp.bfloat16)
        rdma = pltpu.make_async_remote_copy(
            src_ref=comm_ref,
            dst_ref=out_ref.at[pl.ds(my_z * m_per, m_per), :],
            send_sem=send_sem,
            recv_sem=recv_sem,
            device_id=(my_x, my_y, peer_z),
            device_id_type=pl.DeviceIdType.MESH,
        )
        rdma.start()

        dma_mine.wait()
        local_bf[:, :] = xv_mine[:, :].astype(jnp.bfloat16)
        dma_out = pltpu.make_async_copy(
            local_bf, out_ref.at[pl.ds(my_z * m_per, m_per), :], out_sem
        )
        dma_out.start()
        dma_out.wait()

        rdma.wait()

    return pl.pallas_call(
        body,
        out_shape=jax.ShapeDtypeStruct((2 * m_per, n_half), jnp.bfloat16),
        in_specs=[pl.BlockSpec(memory_space=pl.ANY)],
        out_specs=pl.BlockSpec(memory_space=pl.ANY),
        scratch_shapes=[
            pltpu.VMEM((m_per, n_half), jnp.float32),
            pltpu.VMEM((m_per, n_half), jnp.float32),
            pltpu.VMEM((m_per, n_half), jnp.bfloat16),
            pltpu.VMEM((m_per, n_half), jnp.bfloat16),
            pltpu.SemaphoreType.DMA((2,)),
            pltpu.SemaphoreType.DMA,
            pltpu.SemaphoreType.DMA,
            pltpu.SemaphoreType.DMA,
        ],
        compiler_params=pltpu.CompilerParams(collective_id=0),
    )(x)
